# baseline (speedup 1.0000x reference)
import functools

import jax
import jax.numpy as jnp
from jax import lax
from jax.experimental import pallas as pl
from jax.experimental.pallas import tpu as pltpu
from jax.experimental.pallas import tpu_sc as plsc

EMB = 16


def _xprep(xt):
    R, C = xt.shape
    W = 1024
    pad = 256 - R

    def body(a_ref, o1_ref, o2_ref):
        k = a_ref[...]
        e = jnp.bitwise_and(k, _DW - 1)
        row = (k - e) | ((e & (_DS - 1)) << 3) | (e >> _DSH)
        f = row.astype(jnp.float32)
        ri = lax.broadcasted_iota(jnp.int32, (128, 128), 0)
        ci = lax.broadcasted_iota(jnp.int32, (128, 128), 1)
        eye = (ri == ci).astype(jnp.float32)
        ap2 = jnp.concatenate(
            [f[128:R, :], jnp.zeros((pad, W), jnp.float32)], axis=0)
        o1_ref[...] = jax.lax.dot_general(
            f[0:128, :], eye, (((0,), (0,)), ((), ())),
            preferred_element_type=jnp.float32).astype(jnp.int32)
        o2_ref[...] = jax.lax.dot_general(
            ap2, eye, (((0,), (0,)), ((), ())),
            preferred_element_type=jnp.float32).astype(jnp.int32)

    return pl.pallas_call(
        body, grid=(C // W,),
        in_specs=[pl.BlockSpec((R, W), lambda i: (0, i))],
        out_specs=[pl.BlockSpec((W, 128), lambda i: (i, 0)),
                   pl.BlockSpec((W, 128), lambda i: (i, 0))],
        out_shape=[jax.ShapeDtypeStruct((C, 128), jnp.int32),
                   jax.ShapeDtypeStruct((C, 128), jnp.int32)],
    )(xt)


_DW = 16384
_DS = _DW // 8
_DSH = 11


def _detile16(at):
    R, C = at.shape
    G = 128 // R
    grid = (C + _DW - 1) // _DW

    def body(a_ref, o_ref):
        ri = lax.broadcasted_iota(jnp.int32, (128, 128), 0)
        ci = lax.broadcasted_iota(jnp.int32, (128, 128), 1)
        eye = (ri == ci).astype(jnp.float32)

        def compute(a):
            ap = jnp.concatenate(
                [a[:, g * _DS:(g + 1) * _DS] for g in range(G)], axis=0)
            return jax.lax.dot_general(
                ap, eye, (((0,), (0,)), ((), ())),
                preferred_element_type=jnp.float32)

        pid = pl.program_id(0)

        @pl.when(pid != grid - 1)
        def _full():
            o_ref[...] = compute(a_ref[...])

        @pl.when(pid == grid - 1)
        def _tail():
            cols = lax.broadcasted_iota(jnp.int32, (R, _DW), 1) + pid * _DW
            a = jnp.where(cols < C, a_ref[...], 0.0)
            o_ref[...] = compute(a)

    return pl.pallas_call(
        body, grid=(grid,),
        in_specs=[pl.BlockSpec((R, _DW), lambda i: (0, i))],
        out_specs=pl.BlockSpec((_DS, 128), lambda i: (i, 0)),
        out_shape=jax.ShapeDtypeStruct((grid * _DS, 128), at.dtype),
    )(at)


def _pooled_sum_sc(xt1, xt2, emb, L):
    B = xt1.shape[0]
    V, E = emb.shape
    assert E == EMB
    H0 = 128
    H1 = L - H0

    info = plsc.get_sparse_core_info()
    NC, NS = info.num_cores, info.num_subcores
    NW = NC * NS
    rows_per_w = B // NW
    CB = 8
    n_chunks = rows_per_w // CB
    gather_n = CB * L

    mesh = plsc.VectorSubcoreMesh(core_axis_name="c", subcore_axis_name="s")

    UNROLL = 40
    n_red = L // UNROLL

    @functools.partial(
        pl.kernel,
        out_type=jax.ShapeDtypeStruct((B, EMB), jnp.float32),
        mesh=mesh,
        scratch_types=[
            pltpu.VMEM((CB, 128), jnp.int32),
            pltpu.VMEM((CB, 128), jnp.int32),
            pltpu.VMEM((CB, 128), jnp.int32),
            pltpu.VMEM((CB, 128), jnp.int32),
            pltpu.VMEM((gather_n, EMB), jnp.float32),
            pltpu.VMEM((gather_n, EMB), jnp.float32),
            pltpu.VMEM((CB, EMB), jnp.float32),
            pltpu.SemaphoreType.DMA,
            pltpu.SemaphoreType.DMA,
        ],
        compiler_params=pltpu.CompilerParams(use_tc_tiling_on_sc=False),
    )
    def sc_kernel(xt1_hbm, xt2_hbm, emb_hbm, out_hbm,
                  i1a, i1b, i2a, i2b, rows_a, rows_b, pooled_v,
                  sem_a, sem_b):
        wid = lax.axis_index("s") * NC + lax.axis_index("c")
        row0 = wid * rows_per_w

        def stage_idx(k, idx1_ref, idx2_ref):
            sl = pl.ds(row0 + k * CB, CB)
            pltpu.sync_copy(xt1_hbm.at[sl], idx1_ref)
            pltpu.sync_copy(xt2_hbm.at[sl], idx2_ref)

        def gather_descs(idx1_ref, idx2_ref, rows_ref, sem):
            descs = []
            for r in range(CB):
                descs.append(pltpu.make_async_copy(
                    emb_hbm.at[idx1_ref.at[r]],
                    rows_ref.at[pl.ds(r * L, H0)], sem))
                descs.append(pltpu.make_async_copy(
                    emb_hbm.at[idx2_ref.at[r, pl.ds(0, H1)]],
                    rows_ref.at[pl.ds(r * L + H0, H1)], sem))
            return descs

        stage_idx(0, i1a, i2a)
        for d in gather_descs(i1a, i2a, rows_a, sem_a):
            d.start()

        bufs = ((i1a, i2a, rows_a, sem_a), (i1b, i2b, rows_b, sem_b))

        def pair_body(g, _):
            for b in range(2):
                k = 2 * g + b
                i1c, i2c, rows_c, sem_c = bufs[b]
                i1n, i2n, rows_n, sem_n = bufs[1 - b]

                @pl.when(k + 1 < n_chunks)
                def _prefetch():
                    stage_idx(k + 1, i1n, i2n)
                    for d in gather_descs(i1n, i2n, rows_n, sem_n):
                        d.start()

                for d in gather_descs(i1c, i2c, rows_c, sem_c):
                    d.wait()

                for r in range(CB):
                    base = r * L

                    def red_body(i, accs, base=base):
                        a0, a1, a2, a3 = accs
                        off = base + i * UNROLL
                        for j in range(UNROLL):
                            v = rows_c[off + j]
                            if j % 4 == 0:
                                a0 = a0 + v
                            elif j % 4 == 1:
                                a1 = a1 + v
                            elif j % 4 == 2:
                                a2 = a2 + v
                            else:
                                a3 = a3 + v
                        return (a0, a1, a2, a3)

                    z = jnp.zeros((EMB,), jnp.float32)
                    a0, a1, a2, a3 = lax.fori_loop(
                        0, n_red, red_body, (z, z, z, z))
                    pooled_v[r] = (a0 + a1) + (a2 + a3)

                pltpu.sync_copy(pooled_v,
                                out_hbm.at[pl.ds(row0 + k * CB, CB)])
            return 0

        lax.fori_loop(0, n_chunks // 2, pair_body, 0)

    return sc_kernel(xt1, xt2, emb)


def _mlp_tc(h2d, W1s, b1, W2, b2):
    Bd8 = h2d.shape[0]
    BLK = 1024
    eye8 = jnp.eye(8, dtype=jnp.float32)
    W1d = jnp.kron(eye8, W1s)
    b1d = jnp.tile(b1, 8).reshape(1, 128)
    W2d = jnp.kron(eye8, W2)
    b2d = jnp.tile(b2, 8).reshape(1, 16)

    def body(h_ref, w1_ref, b1_ref, w2_ref, b2_ref, o_ref):
        z = jnp.dot(h_ref[...], w1_ref[...],
                    preferred_element_type=jnp.float32) + b1_ref[...]
        z = jnp.maximum(z, 0.0)
        o_ref[...] = jnp.dot(z, w2_ref[...],
                             preferred_element_type=jnp.float32) + b2_ref[...]

    out = pl.pallas_call(
        body,
        grid=(Bd8 // BLK,),
        in_specs=[
            pl.BlockSpec((BLK, 128), lambda i: (i, 0)),
            pl.BlockSpec((128, 128), lambda i: (0, 0)),
            pl.BlockSpec((1, 128), lambda i: (0, 0)),
            pl.BlockSpec((128, EMB), lambda i: (0, 0)),
            pl.BlockSpec((1, EMB), lambda i: (0, 0)),
        ],
        out_specs=pl.BlockSpec((BLK, EMB), lambda i: (i, 0)),
        out_shape=jax.ShapeDtypeStruct((Bd8, EMB), jnp.float32),
    )(h2d, W1d, b1d, W2d, b2d)
    return out.reshape(Bd8 * 8, 2)


def kernel(x, emb, W1, b1, W2, b2):
    B, L = x.shape
    emb_lin = _detile16(emb.T)
    Vp = emb_lin.shape[0] * 128 // EMB
    xt1, xt2 = _xprep(x.T.astype(jnp.int32))
    pooled = _pooled_sum_sc(xt1, xt2, emb_lin.reshape(Vp, EMB), L)
    return _mlp_tc(pooled.reshape(B // 8, 128),
                   W1 * (1.0 / L), b1, W2, b2)


# speedup vs baseline: 1.1127x; 1.1127x over previous
import functools

import jax
import jax.numpy as jnp
from jax import lax
from jax.experimental import pallas as pl
from jax.experimental.pallas import tpu as pltpu
from jax.experimental.pallas import tpu_sc as plsc

EMB = 16


def _xprep(xt):
    R, C = xt.shape
    W = 1024
    pad = 256 - R

    def body(a_ref, o1_ref, o2_ref):
        k = a_ref[...]
        e = jnp.bitwise_and(k, _DW - 1)
        row = (k - e) | ((e & (_DS - 1)) << 3) | (e >> _DSH)
        f = row.astype(jnp.float32)
        ri = lax.broadcasted_iota(jnp.int32, (128, 128), 0)
        ci = lax.broadcasted_iota(jnp.int32, (128, 128), 1)
        eye = (ri == ci).astype(jnp.float32)
        ap2 = jnp.concatenate(
            [f[128:R, :], jnp.zeros((pad, W), jnp.float32)], axis=0)
        o1_ref[...] = jax.lax.dot_general(
            f[0:128, :], eye, (((0,), (0,)), ((), ())),
            precision=jax.lax.Precision.HIGHEST,
            preferred_element_type=jnp.float32).astype(jnp.int32)
        o2_ref[...] = jax.lax.dot_general(
            ap2, eye, (((0,), (0,)), ((), ())),
            precision=jax.lax.Precision.HIGHEST,
            preferred_element_type=jnp.float32).astype(jnp.int32)

    return pl.pallas_call(
        body, grid=(C // W,),
        in_specs=[pl.BlockSpec((R, W), lambda i: (0, i))],
        out_specs=[pl.BlockSpec((W, 128), lambda i: (i, 0)),
                   pl.BlockSpec((W, 128), lambda i: (i, 0))],
        out_shape=[jax.ShapeDtypeStruct((C, 128), jnp.int32),
                   jax.ShapeDtypeStruct((C, 128), jnp.int32)],
    )(xt)


_DW = 16384
_DS = _DW // 8
_DSH = 11


def _detile16(at):
    R, C = at.shape
    G = 128 // R
    grid = (C + _DW - 1) // _DW

    def body(a_ref, o_ref):
        ri = lax.broadcasted_iota(jnp.int32, (128, 128), 0)
        ci = lax.broadcasted_iota(jnp.int32, (128, 128), 1)
        eye = (ri == ci).astype(jnp.float32)

        def compute(a):
            ap = jnp.concatenate(
                [a[:, g * _DS:(g + 1) * _DS] for g in range(G)], axis=0)
            return jax.lax.dot_general(
                ap, eye, (((0,), (0,)), ((), ())),
                precision=jax.lax.Precision.HIGHEST,
                preferred_element_type=jnp.float32)

        pid = pl.program_id(0)

        @pl.when(pid != grid - 1)
        def _full():
            o_ref[...] = compute(a_ref[...])

        @pl.when(pid == grid - 1)
        def _tail():
            cols = lax.broadcasted_iota(jnp.int32, (R, _DW), 1) + pid * _DW
            a = jnp.where(cols < C, a_ref[...], 0.0)
            o_ref[...] = compute(a)

    return pl.pallas_call(
        body, grid=(grid,),
        in_specs=[pl.BlockSpec((R, _DW), lambda i: (0, i))],
        out_specs=pl.BlockSpec((_DS, 128), lambda i: (i, 0)),
        out_shape=jax.ShapeDtypeStruct((grid * _DS, 128), at.dtype),
    )(at)


def _pooled_sum_sc(xt1, xt2, emb, L):
    B = xt1.shape[0]
    V, E = emb.shape
    assert E == EMB
    H0 = 128
    H1 = L - H0

    info = plsc.get_sparse_core_info()
    NC, NS = info.num_cores, info.num_subcores
    NW = NC * NS
    rows_per_w = B // NW
    CB = 8
    n_chunks = rows_per_w // CB
    gather_n = CB * L

    mesh = plsc.VectorSubcoreMesh(core_axis_name="c", subcore_axis_name="s")

    UNROLL = 40
    n_red = L // UNROLL

    @functools.partial(
        pl.kernel,
        out_type=jax.ShapeDtypeStruct((B, EMB), jnp.float32),
        mesh=mesh,
        scratch_types=[
            pltpu.VMEM((CB, 128), jnp.int32),
            pltpu.VMEM((CB, 128), jnp.int32),
            pltpu.VMEM((CB, 128), jnp.int32),
            pltpu.VMEM((CB, 128), jnp.int32),
            pltpu.VMEM((gather_n, EMB), jnp.float32),
            pltpu.VMEM((gather_n, EMB), jnp.float32),
            pltpu.VMEM((CB, EMB), jnp.float32),
            pltpu.SemaphoreType.DMA,
            pltpu.SemaphoreType.DMA,
        ],
        compiler_params=pltpu.CompilerParams(use_tc_tiling_on_sc=False),
    )
    def sc_kernel(xt1_hbm, xt2_hbm, emb_hbm, out_hbm,
                  i1a, i1b, i2a, i2b, rows_a, rows_b, pooled_v,
                  sem_a, sem_b):
        wid = lax.axis_index("s") * NC + lax.axis_index("c")
        row0 = wid * rows_per_w

        def stage_idx(k, idx1_ref, idx2_ref):
            sl = pl.ds(row0 + k * CB, CB)
            pltpu.sync_copy(xt1_hbm.at[sl], idx1_ref)
            pltpu.sync_copy(xt2_hbm.at[sl], idx2_ref)

        def gather_descs(idx1_ref, idx2_ref, rows_ref, sem):
            descs = []
            for r in range(CB):
                descs.append(pltpu.make_async_copy(
                    emb_hbm.at[idx1_ref.at[r]],
                    rows_ref.at[pl.ds(r * L, H0)], sem))
                descs.append(pltpu.make_async_copy(
                    emb_hbm.at[idx2_ref.at[r, pl.ds(0, H1)]],
                    rows_ref.at[pl.ds(r * L + H0, H1)], sem))
            return descs

        stage_idx(0, i1a, i2a)
        for d in gather_descs(i1a, i2a, rows_a, sem_a):
            d.start()

        bufs = ((i1a, i2a, rows_a, sem_a), (i1b, i2b, rows_b, sem_b))

        def pair_body(g, _):
            for b in range(2):
                k = 2 * g + b
                i1c, i2c, rows_c, sem_c = bufs[b]
                i1n, i2n, rows_n, sem_n = bufs[1 - b]

                @pl.when(k + 1 < n_chunks)
                def _prefetch():
                    stage_idx(k + 1, i1n, i2n)
                    for d in gather_descs(i1n, i2n, rows_n, sem_n):
                        d.start()

                for d in gather_descs(i1c, i2c, rows_c, sem_c):
                    d.wait()

                for r in range(CB):
                    base = r * L

                    def red_body(i, accs, base=base):
                        a0, a1, a2, a3 = accs
                        off = base + i * UNROLL
                        for j in range(UNROLL):
                            v = rows_c[off + j]
                            if j % 4 == 0:
                                a0 = a0 + v
                            elif j % 4 == 1:
                                a1 = a1 + v
                            elif j % 4 == 2:
                                a2 = a2 + v
                            else:
                                a3 = a3 + v
                        return (a0, a1, a2, a3)

                    z = jnp.zeros((EMB,), jnp.float32)
                    a0, a1, a2, a3 = lax.fori_loop(
                        0, n_red, red_body, (z, z, z, z))
                    pooled_v[r] = (a0 + a1) + (a2 + a3)

                pltpu.sync_copy(pooled_v,
                                out_hbm.at[pl.ds(row0 + k * CB, CB)])
            return 0

        lax.fori_loop(0, n_chunks // 2, pair_body, 0)

    return sc_kernel(xt1, xt2, emb)


def _mlp_tc(h2d, W1s, b1, W2, b2):
    Bd8 = h2d.shape[0]
    BLK = 1024
    eye8 = jnp.eye(8, dtype=jnp.float32)
    W1d = jnp.kron(eye8, W1s)
    b1d = jnp.tile(b1, 8).reshape(1, 128)
    W2d = jnp.kron(eye8, W2)
    b2d = jnp.tile(b2, 8).reshape(1, 16)

    def body(h_ref, w1_ref, b1_ref, w2_ref, b2_ref, o_ref):
        z = jnp.dot(h_ref[...], w1_ref[...],
                    preferred_element_type=jnp.float32) + b1_ref[...]
        z = jnp.maximum(z, 0.0)
        o_ref[...] = jnp.dot(z, w2_ref[...],
                             preferred_element_type=jnp.float32) + b2_ref[...]

    out = pl.pallas_call(
        body,
        grid=(Bd8 // BLK,),
        in_specs=[
            pl.BlockSpec((BLK, 128), lambda i: (i, 0)),
            pl.BlockSpec((128, 128), lambda i: (0, 0)),
            pl.BlockSpec((1, 128), lambda i: (0, 0)),
            pl.BlockSpec((128, EMB), lambda i: (0, 0)),
            pl.BlockSpec((1, EMB), lambda i: (0, 0)),
        ],
        out_specs=pl.BlockSpec((BLK, EMB), lambda i: (i, 0)),
        out_shape=jax.ShapeDtypeStruct((Bd8, EMB), jnp.float32),
    )(h2d, W1d, b1d, W2d, b2d)
    return out.reshape(Bd8 * 8, 2)


def kernel(x, emb, W1, b1, W2, b2):
    B, L = x.shape
    emb_lin = _detile16(emb.T)
    Vp = emb_lin.shape[0] * 128 // EMB
    xt1, xt2 = _xprep(x.T.astype(jnp.int32))
    pooled = _pooled_sum_sc(xt1, xt2, emb_lin.reshape(Vp, EMB), L)
    return _mlp_tc(pooled.reshape(B // 8, 128),
                   W1 * (1.0 / L), b1, W2, b2)


# speedup vs baseline: 1.1842x; 1.0643x over previous
import functools

import jax
import jax.numpy as jnp
from jax import lax
from jax.experimental import pallas as pl
from jax.experimental.pallas import tpu as pltpu
from jax.experimental.pallas import tpu_sc as plsc

EMB = 16


def _xprep(xt):
    R, C = xt.shape
    W = 1024
    pad = 256 - R

    def body(a_ref, o1_ref, o2_ref):
        k = a_ref[...]
        e = jnp.bitwise_and(k, _DW - 1)
        row = (k - e) | ((e & (_DS - 1)) << 3) | (e >> _DSH)
        f = row.astype(jnp.float32)
        ri = lax.broadcasted_iota(jnp.int32, (128, 128), 0)
        ci = lax.broadcasted_iota(jnp.int32, (128, 128), 1)
        eye = (ri == ci).astype(jnp.float32)
        ap2 = jnp.concatenate(
            [f[128:R, :], jnp.zeros((pad, W), jnp.float32)], axis=0)
        o1_ref[...] = jax.lax.dot_general(
            f[0:128, :], eye, (((0,), (0,)), ((), ())),
            precision=jax.lax.Precision.HIGHEST,
            preferred_element_type=jnp.float32).astype(jnp.int32)
        o2_ref[...] = jax.lax.dot_general(
            ap2, eye, (((0,), (0,)), ((), ())),
            precision=jax.lax.Precision.HIGHEST,
            preferred_element_type=jnp.float32).astype(jnp.int32)

    return pl.pallas_call(
        body, grid=(C // W,),
        in_specs=[pl.BlockSpec((R, W), lambda i: (0, i))],
        out_specs=[pl.BlockSpec((W, 128), lambda i: (i, 0)),
                   pl.BlockSpec((W, 128), lambda i: (i, 0))],
        out_shape=[jax.ShapeDtypeStruct((C, 128), jnp.int32),
                   jax.ShapeDtypeStruct((C, 128), jnp.int32)],
    )(xt)


_DW = 16384
_DS = _DW // 8
_DSH = 11


def _detile16(at):
    R, C = at.shape
    G = 128 // R
    grid = (C + _DW - 1) // _DW

    def body(a_ref, o_ref):
        ri = lax.broadcasted_iota(jnp.int32, (128, 128), 0)
        ci = lax.broadcasted_iota(jnp.int32, (128, 128), 1)
        eye = (ri == ci).astype(jnp.float32)

        def compute(a):
            ap = jnp.concatenate(
                [a[:, g * _DS:(g + 1) * _DS] for g in range(G)], axis=0)
            return jax.lax.dot_general(
                ap, eye, (((0,), (0,)), ((), ())),
                preferred_element_type=jnp.float32)

        pid = pl.program_id(0)

        @pl.when(pid != grid - 1)
        def _full():
            o_ref[...] = compute(a_ref[...])

        @pl.when(pid == grid - 1)
        def _tail():
            cols = lax.broadcasted_iota(jnp.int32, (R, _DW), 1) + pid * _DW
            a = jnp.where(cols < C, a_ref[...], 0.0)
            o_ref[...] = compute(a)

    return pl.pallas_call(
        body, grid=(grid,),
        in_specs=[pl.BlockSpec((R, _DW), lambda i: (0, i))],
        out_specs=pl.BlockSpec((_DS, 128), lambda i: (i, 0)),
        out_shape=jax.ShapeDtypeStruct((grid * _DS, 128), at.dtype),
    )(at)


def _pooled_sum_sc(xt1, xt2, emb, L):
    B = xt1.shape[0]
    V, E = emb.shape
    assert E == EMB
    H0 = 128
    H1 = L - H0

    info = plsc.get_sparse_core_info()
    NC, NS = info.num_cores, info.num_subcores
    NW = NC * NS
    rows_per_w = B // NW
    CB = 8
    n_chunks = rows_per_w // CB
    gather_n = CB * L

    mesh = plsc.VectorSubcoreMesh(core_axis_name="c", subcore_axis_name="s")

    UNROLL = 40
    n_red = L // UNROLL

    @functools.partial(
        pl.kernel,
        out_type=jax.ShapeDtypeStruct((B, EMB), jnp.float32),
        mesh=mesh,
        scratch_types=[
            pltpu.VMEM((CB, 128), jnp.int32),
            pltpu.VMEM((CB, 128), jnp.int32),
            pltpu.VMEM((CB, 128), jnp.int32),
            pltpu.VMEM((CB, 128), jnp.int32),
            pltpu.VMEM((gather_n, EMB), jnp.float32),
            pltpu.VMEM((gather_n, EMB), jnp.float32),
            pltpu.VMEM((CB, EMB), jnp.float32),
            pltpu.SemaphoreType.DMA,
            pltpu.SemaphoreType.DMA,
        ],
        compiler_params=pltpu.CompilerParams(use_tc_tiling_on_sc=False),
    )
    def sc_kernel(xt1_hbm, xt2_hbm, emb_hbm, out_hbm,
                  i1a, i1b, i2a, i2b, rows_a, rows_b, pooled_v,
                  sem_a, sem_b):
        wid = lax.axis_index("s") * NC + lax.axis_index("c")
        row0 = wid * rows_per_w

        def stage_idx(k, idx1_ref, idx2_ref):
            sl = pl.ds(row0 + k * CB, CB)
            pltpu.sync_copy(xt1_hbm.at[sl], idx1_ref)
            pltpu.sync_copy(xt2_hbm.at[sl], idx2_ref)

        def gather_descs(idx1_ref, idx2_ref, rows_ref, sem):
            descs = []
            for r in range(CB):
                descs.append(pltpu.make_async_copy(
                    emb_hbm.at[idx1_ref.at[r]],
                    rows_ref.at[pl.ds(r * L, H0)], sem))
                descs.append(pltpu.make_async_copy(
                    emb_hbm.at[idx2_ref.at[r, pl.ds(0, H1)]],
                    rows_ref.at[pl.ds(r * L + H0, H1)], sem))
            return descs

        stage_idx(0, i1a, i2a)
        for d in gather_descs(i1a, i2a, rows_a, sem_a):
            d.start()

        bufs = ((i1a, i2a, rows_a, sem_a), (i1b, i2b, rows_b, sem_b))

        def pair_body(g, _):
            for b in range(2):
                k = 2 * g + b
                i1c, i2c, rows_c, sem_c = bufs[b]
                i1n, i2n, rows_n, sem_n = bufs[1 - b]

                @pl.when(k + 1 < n_chunks)
                def _prefetch():
                    stage_idx(k + 1, i1n, i2n)
                    for d in gather_descs(i1n, i2n, rows_n, sem_n):
                        d.start()

                for d in gather_descs(i1c, i2c, rows_c, sem_c):
                    d.wait()

                for r in range(CB):
                    base = r * L

                    def red_body(i, accs, base=base):
                        a0, a1, a2, a3 = accs
                        off = base + i * UNROLL
                        for j in range(UNROLL):
                            v = rows_c[off + j]
                            if j % 4 == 0:
                                a0 = a0 + v
                            elif j % 4 == 1:
                                a1 = a1 + v
                            elif j % 4 == 2:
                                a2 = a2 + v
                            else:
                                a3 = a3 + v
                        return (a0, a1, a2, a3)

                    z = jnp.zeros((EMB,), jnp.float32)
                    a0, a1, a2, a3 = lax.fori_loop(
                        0, n_red, red_body, (z, z, z, z))
                    pooled_v[r] = (a0 + a1) + (a2 + a3)

                pltpu.sync_copy(pooled_v,
                                out_hbm.at[pl.ds(row0 + k * CB, CB)])
            return 0

        lax.fori_loop(0, n_chunks // 2, pair_body, 0)

    return sc_kernel(xt1, xt2, emb)


def _mlp_tc(h2d, W1s, b1, W2, b2):
    Bd8 = h2d.shape[0]
    BLK = 1024
    eye8 = jnp.eye(8, dtype=jnp.float32)
    W1d = jnp.kron(eye8, W1s)
    b1d = jnp.tile(b1, 8).reshape(1, 128)
    W2d = jnp.kron(eye8, W2)
    b2d = jnp.tile(b2, 8).reshape(1, 16)

    def body(h_ref, w1_ref, b1_ref, w2_ref, b2_ref, o_ref):
        z = jnp.dot(h_ref[...], w1_ref[...],
                    preferred_element_type=jnp.float32) + b1_ref[...]
        z = jnp.maximum(z, 0.0)
        o_ref[...] = jnp.dot(z, w2_ref[...],
                             preferred_element_type=jnp.float32) + b2_ref[...]

    out = pl.pallas_call(
        body,
        grid=(Bd8 // BLK,),
        in_specs=[
            pl.BlockSpec((BLK, 128), lambda i: (i, 0)),
            pl.BlockSpec((128, 128), lambda i: (0, 0)),
            pl.BlockSpec((1, 128), lambda i: (0, 0)),
            pl.BlockSpec((128, EMB), lambda i: (0, 0)),
            pl.BlockSpec((1, EMB), lambda i: (0, 0)),
        ],
        out_specs=pl.BlockSpec((BLK, EMB), lambda i: (i, 0)),
        out_shape=jax.ShapeDtypeStruct((Bd8, EMB), jnp.float32),
    )(h2d, W1d, b1d, W2d, b2d)
    return out.reshape(Bd8 * 8, 2)


def kernel(x, emb, W1, b1, W2, b2):
    B, L = x.shape
    emb_lin = _detile16(emb.T)
    Vp = emb_lin.shape[0] * 128 // EMB
    xt1, xt2 = _xprep(x.T.astype(jnp.int32))
    pooled = _pooled_sum_sc(xt1, xt2, emb_lin.reshape(Vp, EMB), L)
    return _mlp_tc(pooled.reshape(B // 8, 128),
                   W1 * (1.0 / L), b1, W2, b2)


# speedup vs baseline: 1.2622x; 1.0659x over previous
import functools

import jax
import jax.numpy as jnp
from jax import lax
from jax.experimental import pallas as pl
from jax.experimental.pallas import tpu as pltpu
from jax.experimental.pallas import tpu_sc as plsc

EMB = 16


def _xprep(xt):
    R, C = xt.shape
    W = 1024
    pad = 256 - R

    def body(a_ref, o1_ref, o2_ref):
        k = a_ref[...]
        e = jnp.bitwise_and(k, _DW - 1)
        row = (k - e) | ((e & (_DS - 1)) << 3) | (e >> _DSH)
        f = row.astype(jnp.float32)
        ri = lax.broadcasted_iota(jnp.int32, (128, 128), 0)
        ci = lax.broadcasted_iota(jnp.int32, (128, 128), 1)
        eye = (ri == ci).astype(jnp.float32)
        ap2 = jnp.concatenate(
            [f[128:R, :], jnp.zeros((pad, W), jnp.float32)], axis=0)
        o1_ref[...] = jax.lax.dot_general(
            f[0:128, :], eye, (((0,), (0,)), ((), ())),
            precision=jax.lax.Precision.HIGHEST,
            preferred_element_type=jnp.float32).astype(jnp.int32)
        o2_ref[...] = jax.lax.dot_general(
            ap2, eye, (((0,), (0,)), ((), ())),
            precision=jax.lax.Precision.HIGHEST,
            preferred_element_type=jnp.float32).astype(jnp.int32)

    return pl.pallas_call(
        body, grid=(C // W,),
        in_specs=[pl.BlockSpec((R, W), lambda i: (0, i))],
        out_specs=[pl.BlockSpec((W, 128), lambda i: (i, 0)),
                   pl.BlockSpec((W, 128), lambda i: (i, 0))],
        out_shape=[jax.ShapeDtypeStruct((C, 128), jnp.int32),
                   jax.ShapeDtypeStruct((C, 128), jnp.int32)],
    )(xt)


_DW = 16384
_DS = _DW // 8
_DSH = 11


def _detile16(at):
    R, C = at.shape
    G = 128 // R
    grid = (C + _DW - 1) // _DW

    def body(a_ref, o_ref):
        ri = lax.broadcasted_iota(jnp.int32, (128, 128), 0)
        ci = lax.broadcasted_iota(jnp.int32, (128, 128), 1)
        eye = (ri == ci).astype(jnp.float32)

        def compute(a):
            ap = jnp.concatenate(
                [a[:, g * _DS:(g + 1) * _DS] for g in range(G)], axis=0)
            return jax.lax.dot_general(
                ap, eye, (((0,), (0,)), ((), ())),
                preferred_element_type=jnp.float32)

        pid = pl.program_id(0)

        @pl.when(pid != grid - 1)
        def _full():
            o_ref[...] = compute(a_ref[...])

        @pl.when(pid == grid - 1)
        def _tail():
            cols = lax.broadcasted_iota(jnp.int32, (R, _DW), 1) + pid * _DW
            a = jnp.where(cols < C, a_ref[...], 0.0)
            o_ref[...] = compute(a)

    return pl.pallas_call(
        body, grid=(grid,),
        in_specs=[pl.BlockSpec((R, _DW), lambda i: (0, i))],
        out_specs=pl.BlockSpec((_DS, 128), lambda i: (i, 0)),
        out_shape=jax.ShapeDtypeStruct((grid * _DS, 128), at.dtype),
    )(at)


def _pooled_sum_sc(xt1, xt2, emb, L):
    B = xt1.shape[0]
    V, E = emb.shape
    assert E == EMB
    H0 = 128
    H1 = L - H0

    info = plsc.get_sparse_core_info()
    NC, NS = info.num_cores, info.num_subcores
    NW = NC * NS
    rows_per_w = B // NW
    CB = 8
    n_chunks = rows_per_w // CB
    gather_n = CB * L

    mesh = plsc.VectorSubcoreMesh(core_axis_name="c", subcore_axis_name="s")

    UNROLL = 40
    n_red = L // UNROLL

    @functools.partial(
        pl.kernel,
        out_type=jax.ShapeDtypeStruct((B, EMB), jnp.float32),
        mesh=mesh,
        scratch_types=[
            pltpu.VMEM((CB, 128), jnp.int32),
            pltpu.VMEM((CB, 128), jnp.int32),
            pltpu.VMEM((CB, 128), jnp.int32),
            pltpu.VMEM((CB, 128), jnp.int32),
            pltpu.VMEM((gather_n, EMB), jnp.float32),
            pltpu.VMEM((gather_n, EMB), jnp.float32),
            pltpu.VMEM((CB, EMB), jnp.float32),
            pltpu.SemaphoreType.DMA,
            pltpu.SemaphoreType.DMA,
            pltpu.SemaphoreType.DMA,
        ],
        compiler_params=pltpu.CompilerParams(use_tc_tiling_on_sc=False),
    )
    def sc_kernel(xt1_hbm, xt2_hbm, emb_hbm, out_hbm,
                  i1a, i1b, i2a, i2b, rows_a, rows_b, pooled_v,
                  sem_a, sem_b, isem):
        wid = lax.axis_index("s") * NC + lax.axis_index("c")
        row0 = wid * rows_per_w

        def stage_idx(k, idx1_ref, idx2_ref):
            sl = pl.ds(row0 + k * CB, CB)
            d1 = pltpu.make_async_copy(xt1_hbm.at[sl], idx1_ref, isem)
            d2 = pltpu.make_async_copy(xt2_hbm.at[sl], idx2_ref, isem)
            d1.start()
            d2.start()
            d1.wait()
            d2.wait()

        def gather_descs(idx1_ref, idx2_ref, rows_ref, sem):
            descs = []
            for r in range(CB):
                descs.append(pltpu.make_async_copy(
                    emb_hbm.at[idx1_ref.at[r]],
                    rows_ref.at[pl.ds(r * L, H0)], sem))
                descs.append(pltpu.make_async_copy(
                    emb_hbm.at[idx2_ref.at[r, pl.ds(0, H1)]],
                    rows_ref.at[pl.ds(r * L + H0, H1)], sem))
            return descs

        stage_idx(0, i1a, i2a)
        for d in gather_descs(i1a, i2a, rows_a, sem_a):
            d.start()

        bufs = ((i1a, i2a, rows_a, sem_a), (i1b, i2b, rows_b, sem_b))

        def pair_body(g, _):
            for b in range(2):
                k = 2 * g + b
                i1c, i2c, rows_c, sem_c = bufs[b]
                i1n, i2n, rows_n, sem_n = bufs[1 - b]

                @pl.when(k + 1 < n_chunks)
                def _prefetch():
                    stage_idx(k + 1, i1n, i2n)
                    for d in gather_descs(i1n, i2n, rows_n, sem_n):
                        d.start()

                for d in gather_descs(i1c, i2c, rows_c, sem_c):
                    d.wait()

                for r in range(CB):
                    base = r * L

                    def red_body(i, accs, base=base):
                        a0, a1, a2, a3 = accs
                        off = base + i * UNROLL
                        for j in range(UNROLL):
                            v = rows_c[off + j]
                            if j % 4 == 0:
                                a0 = a0 + v
                            elif j % 4 == 1:
                                a1 = a1 + v
                            elif j % 4 == 2:
                                a2 = a2 + v
                            else:
                                a3 = a3 + v
                        return (a0, a1, a2, a3)

                    z = jnp.zeros((EMB,), jnp.float32)
                    a0, a1, a2, a3 = lax.fori_loop(
                        0, n_red, red_body, (z, z, z, z))
                    pooled_v[r] = (a0 + a1) + (a2 + a3)

                pltpu.sync_copy(pooled_v,
                                out_hbm.at[pl.ds(row0 + k * CB, CB)])
            return 0

        lax.fori_loop(0, n_chunks // 2, pair_body, 0)

    return sc_kernel(xt1, xt2, emb)


def _mlp_tc(h2d, W1s, b1, W2, b2):
    Bd8 = h2d.shape[0]
    BLK = 1024
    eye8 = jnp.eye(8, dtype=jnp.float32)
    W1d = jnp.kron(eye8, W1s)
    b1d = jnp.tile(b1, 8).reshape(1, 128)
    W2d = jnp.kron(eye8, W2)
    b2d = jnp.tile(b2, 8).reshape(1, 16)

    def body(h_ref, w1_ref, b1_ref, w2_ref, b2_ref, o_ref):
        z = jnp.dot(h_ref[...], w1_ref[...],
                    preferred_element_type=jnp.float32) + b1_ref[...]
        z = jnp.maximum(z, 0.0)
        o_ref[...] = jnp.dot(z, w2_ref[...],
                             preferred_element_type=jnp.float32) + b2_ref[...]

    out = pl.pallas_call(
        body,
        grid=(Bd8 // BLK,),
        in_specs=[
            pl.BlockSpec((BLK, 128), lambda i: (i, 0)),
            pl.BlockSpec((128, 128), lambda i: (0, 0)),
            pl.BlockSpec((1, 128), lambda i: (0, 0)),
            pl.BlockSpec((128, EMB), lambda i: (0, 0)),
            pl.BlockSpec((1, EMB), lambda i: (0, 0)),
        ],
        out_specs=pl.BlockSpec((BLK, EMB), lambda i: (i, 0)),
        out_shape=jax.ShapeDtypeStruct((Bd8, EMB), jnp.float32),
    )(h2d, W1d, b1d, W2d, b2d)
    return out.reshape(Bd8 * 8, 2)


def kernel(x, emb, W1, b1, W2, b2):
    B, L = x.shape
    emb_lin = _detile16(emb.T)
    Vp = emb_lin.shape[0] * 128 // EMB
    xt1, xt2 = _xprep(x.T.astype(jnp.int32))
    pooled = _pooled_sum_sc(xt1, xt2, emb_lin.reshape(Vp, EMB), L)
    return _mlp_tc(pooled.reshape(B // 8, 128),
                   W1 * (1.0 / L), b1, W2, b2)
